# baseline (speedup 1.0000x reference)
import jax
import jax.numpy as jnp
from jax import lax
from jax.experimental import pallas as pl
from jax.experimental.pallas import tpu as pltpu
from jax.experimental.pallas import tpu_sc as plsc

_L = 16
_ROWS = 128
_N = 32768
_NCH = _N // _L
_UNROLL = 8
_NW = 32
_SC_ROWS = 96
_RPW = _SC_ROWS // _NW
_TC_ROWS = _ROWS - _SC_ROWS
_TC_BLOCK = 8
_BISECT = 14

_f32 = jnp.float32
_i32 = jnp.int32


def _bsum(v):
    lane = lax.iota(_i32, _L)
    for d in (1, 2, 4, 8):
        v = v + v[lane ^ d]
    return v


def _bmax(v):
    lane = lax.iota(_i32, _L)
    for d in (1, 2, 4, 8):
        v = jnp.maximum(v, v[lane ^ d])
    return v


def _row_compute(xbuf, cand):
    lane = lax.iota(_i32, _L)
    ninf = jnp.full((_L,), -jnp.inf, _f32)

    @plsc.parallel_loop(0, _NCH, unroll=_UNROLL, carry=ninf)
    def mxloop(i, acc):
        return jnp.maximum(acc, xbuf[pl.ds(i * _L, _L)])

    m = _bmax(mxloop)
    lo0 = m - 1.0

    @plsc.parallel_loop(0, _NCH, unroll=_UNROLL, carry=jnp.zeros((_L,), _i32))
    def compact(i, off):
        v = xbuf[pl.ds(i * _L, _L)]
        aliv = v >= lo0
        ai = jnp.where(aliv, 1, 0)
        pos = off + plsc.cumsum(ai) - ai
        plsc.store_scatter(cand, [pos], v, mask=aliv)
        return off + plsc.all_reduce_population_count(aliv)

    ncand = compact
    nvec = (jnp.max(ncand) + (_L - 1)) // _L

    def fsum(mid):
        def body(i, acc):
            v = cand[pl.ds(i * _L, _L)]
            valid = (lane + i * _L) < ncand
            return acc + jnp.where(valid, jnp.maximum(v - mid, 0.0), 0.0)

        return _bsum(lax.fori_loop(0, nvec, body, jnp.zeros((_L,), _f32)))

    def bis(_, carry):
        lo, hi = carry
        mid = 0.5 * (lo + hi)
        pred = fsum(mid) > 1.0
        return jnp.where(pred, mid, lo), jnp.where(pred, hi, mid)

    lo, hi = lax.fori_loop(0, _BISECT, bis, (lo0, m))
    mid = 0.5 * (lo + hi)

    def ksbody(i, acc):
        ka, sa = acc
        v = cand[pl.ds(i * _L, _L)]
        sel = ((lane + i * _L) < ncand) & (v > mid)
        return ka + jnp.where(sel, 1.0, 0.0), sa + jnp.where(sel, v, 0.0)

    z = jnp.zeros((_L,), _f32)
    ka, sa = lax.fori_loop(0, nvec, ksbody, (z, z))
    kp = jnp.maximum(_bsum(ka), 1.0)
    sp = _bsum(sa)
    tau = (sp - 1.0) / kp

    @plsc.parallel_loop(0, _NCH, unroll=_UNROLL)
    def outp(i):
        base = i * _L
        xbuf[pl.ds(base, _L)] = jnp.maximum(xbuf[pl.ds(base, _L)] + tau, 0.0)


def _sc_body(x_hbm, out_hbm, xb0, xb1, cand, si0, si1, so0, so1):
    wid = lax.axis_index("s") * 2 + lax.axis_index("c")
    row0 = wid * _RPW
    bufs = (xb0, xb1)
    isems = (si0, si1)
    osems = (so0, so1)

    h_in = pltpu.async_copy(x_hbm.at[row0], bufs[0], isems[0])
    h_out = [None, None]
    for r in range(_RPW):
        cur = r % 2
        nxt = (r + 1) % 2
        if r + 1 < _RPW:
            if h_out[nxt] is not None:
                h_out[nxt].wait()
                h_out[nxt] = None
            h_next = pltpu.async_copy(x_hbm.at[row0 + r + 1], bufs[nxt], isems[nxt])
        h_in.wait()
        _row_compute(bufs[cur], cand)
        h_out[cur] = pltpu.async_copy(bufs[cur], out_hbm.at[row0 + r], osems[cur])
        if r + 1 < _RPW:
            h_in = h_next
    for h in h_out:
        if h is not None:
            h.wait()


def _make(interpret=False):
    return pl.kernel(
        _sc_body,
        out_type=jax.ShapeDtypeStruct((_SC_ROWS, _N), _f32),
        mesh=plsc.VectorSubcoreMesh(
            core_axis_name="c", subcore_axis_name="s", num_cores=2, num_subcores=16
        ),
        scratch_types=[
            pltpu.VMEM((_N,), _f32),
            pltpu.VMEM((_N,), _f32),
            pltpu.VMEM((_N,), _f32),
            pltpu.SemaphoreType.DMA,
            pltpu.SemaphoreType.DMA,
            pltpu.SemaphoreType.DMA,
            pltpu.SemaphoreType.DMA,
        ],
        compiler_params=pltpu.CompilerParams(needs_layout_passes=False),
        interpret=interpret,
    )


_sc_sparsemax = _make()


def _tc_block(x_ref, o_ref):
    x = x_ref[...]
    m = jnp.max(x, axis=-1, keepdims=True)
    lo = m - 1.0
    hi = m

    def body(_, carry):
        lo, hi = carry
        mid = 0.5 * (lo + hi)
        s = jnp.sum(jnp.maximum(x - mid, 0.0), axis=-1, keepdims=True)
        pred = s > 1.0
        return jnp.where(pred, mid, lo), jnp.where(pred, hi, mid)

    lo, hi = lax.fori_loop(0, _BISECT, body, (lo, hi))
    mid = 0.5 * (lo + hi)
    mask = x > mid
    k = jnp.maximum(jnp.sum(mask.astype(_f32), axis=-1, keepdims=True), 1.0)
    s = jnp.sum(jnp.where(mask, x, 0.0), axis=-1, keepdims=True)
    tau = (s - 1.0) / k
    o_ref[...] = jnp.maximum(x + tau, 0.0)


def _tc_sparsemax(x):
    rows, n = x.shape
    return pl.pallas_call(
        _tc_block,
        grid=(rows // _TC_BLOCK,),
        in_specs=[pl.BlockSpec((_TC_BLOCK, n), lambda i: (i, 0))],
        out_specs=pl.BlockSpec((_TC_BLOCK, n), lambda i: (i, 0)),
        out_shape=jax.ShapeDtypeStruct((rows, n), x.dtype),
    )(x)


@jax.jit
def kernel(x):
    sc_out = _sc_sparsemax(x[:_SC_ROWS])
    tc_out = _tc_sparsemax(x[_SC_ROWS:])
    return jnp.concatenate([sc_out, tc_out], axis=0)


# speedup vs baseline: 1.3058x; 1.3058x over previous
import jax
import jax.numpy as jnp
from jax import lax
from jax.experimental import pallas as pl
from jax.experimental.pallas import tpu as pltpu
from jax.experimental.pallas import tpu_sc as plsc

_L = 16
_ROWS = 128
_N = 32768
_NCH = _N // _L
_UNROLL = 8
_NW = 32
_RPW = _ROWS // _NW
_BISECT = 14

_f32 = jnp.float32
_i32 = jnp.int32


def _bsum(v):
    lane = lax.iota(_i32, _L)
    for d in (1, 2, 4, 8):
        v = v + v[lane ^ d]
    return v


def _bmax(v):
    lane = lax.iota(_i32, _L)
    for d in (1, 2, 4, 8):
        v = jnp.maximum(v, v[lane ^ d])
    return v


def _row_compute(xbuf, cand):
    lane = lax.iota(_i32, _L)
    ninf = jnp.full((_L,), -jnp.inf, _f32)

    @plsc.parallel_loop(0, _NCH, unroll=_UNROLL, carry=ninf)
    def mxloop(i, acc):
        return jnp.maximum(acc, xbuf[pl.ds(i * _L, _L)])

    m = _bmax(mxloop)
    lo0 = m - 1.0

    @plsc.parallel_loop(0, _NCH, unroll=_UNROLL, carry=jnp.zeros((_L,), _i32))
    def compact(i, off):
        v = xbuf[pl.ds(i * _L, _L)]
        aliv = v >= lo0
        ai = jnp.where(aliv, 1, 0)
        pos = off + plsc.cumsum(ai) - ai
        plsc.store_scatter(cand, [pos], v, mask=aliv)
        return off + plsc.all_reduce_population_count(aliv)

    ncand = compact
    nvec = (jnp.max(ncand) + (_L - 1)) // _L

    def fsum(mid):
        def body(i, acc):
            v = cand[pl.ds(i * _L, _L)]
            valid = (lane + i * _L) < ncand
            return acc + jnp.where(valid, jnp.maximum(v - mid, 0.0), 0.0)

        return _bsum(lax.fori_loop(0, nvec, body, jnp.zeros((_L,), _f32)))

    def bis(_, carry):
        lo, hi = carry
        mid = 0.5 * (lo + hi)
        pred = fsum(mid) > 1.0
        return jnp.where(pred, mid, lo), jnp.where(pred, hi, mid)

    lo, hi = lax.fori_loop(0, _BISECT, bis, (lo0, m))
    mid = 0.5 * (lo + hi)

    def ksbody(i, acc):
        ka, sa = acc
        v = cand[pl.ds(i * _L, _L)]
        sel = ((lane + i * _L) < ncand) & (v > mid)
        return ka + jnp.where(sel, 1.0, 0.0), sa + jnp.where(sel, v, 0.0)

    z = jnp.zeros((_L,), _f32)
    ka, sa = lax.fori_loop(0, nvec, ksbody, (z, z))
    kp = jnp.maximum(_bsum(ka), 1.0)
    sp = _bsum(sa)
    tau = (sp - 1.0) / kp

    @plsc.parallel_loop(0, _NCH, unroll=_UNROLL)
    def outp(i):
        base = i * _L
        xbuf[pl.ds(base, _L)] = jnp.maximum(xbuf[pl.ds(base, _L)] + tau, 0.0)


def _sc_body(x_hbm, out_hbm, xb0, xb1, cand, si0, si1, so0, so1):
    wid = lax.axis_index("s") * 2 + lax.axis_index("c")
    row0 = wid * _RPW
    bufs = (xb0, xb1)
    isems = (si0, si1)
    osems = (so0, so1)

    h_in = pltpu.async_copy(x_hbm.at[row0], bufs[0], isems[0])
    h_out = [None, None]
    for r in range(_RPW):
        cur = r % 2
        nxt = (r + 1) % 2
        if r + 1 < _RPW:
            if h_out[nxt] is not None:
                h_out[nxt].wait()
                h_out[nxt] = None
            h_next = pltpu.async_copy(x_hbm.at[row0 + r + 1], bufs[nxt], isems[nxt])
        h_in.wait()
        _row_compute(bufs[cur], cand)
        h_out[cur] = pltpu.async_copy(bufs[cur], out_hbm.at[row0 + r], osems[cur])
        if r + 1 < _RPW:
            h_in = h_next
    for h in h_out:
        if h is not None:
            h.wait()


def _make(interpret=False):
    return pl.kernel(
        _sc_body,
        out_type=jax.ShapeDtypeStruct((_ROWS, _N), _f32),
        mesh=plsc.VectorSubcoreMesh(
            core_axis_name="c", subcore_axis_name="s", num_cores=2, num_subcores=16
        ),
        scratch_types=[
            pltpu.VMEM((_N,), _f32),
            pltpu.VMEM((_N,), _f32),
            pltpu.VMEM((_N,), _f32),
            pltpu.SemaphoreType.DMA,
            pltpu.SemaphoreType.DMA,
            pltpu.SemaphoreType.DMA,
            pltpu.SemaphoreType.DMA,
        ],
        compiler_params=pltpu.CompilerParams(needs_layout_passes=False),
        interpret=interpret,
    )


_sc_sparsemax = _make()


@jax.jit
def kernel(x):
    return _sc_sparsemax(x)
